# baseline (speedup 1.0000x reference)
import functools

import jax
import jax.numpy as jnp
from jax import lax
from jax.experimental import pallas as pl
from jax.experimental.pallas import tpu as pltpu
from jax.experimental.pallas import tpu_sc as plsc

NC = 2
NS = 16
NW = NC * NS
LANES = 16
KD = 128


def _mesh():
    return plsc.VectorSubcoreMesh(
        core_axis_name="c", subcore_axis_name="s", num_cores=NC, num_subcores=NS
    )


def _deg_body(n_acc, ep, dst3, out, didx, ones_v, zvec, deg_acc):
    c = lax.axis_index("c")
    s = lax.axis_index("s")
    wid = c * NS + s
    rp = n_acc // NS

    pltpu.sync_copy(dst3.at[wid], didx)

    def fill(i, _):
        zvec[pl.ds(i * LANES, LANES)] = jnp.zeros((LANES,), jnp.float32)
        ones_v[pl.ds((i % (KD // LANES)) * LANES, LANES)] = jnp.ones(
            (LANES,), jnp.float32
        )
        return 0

    lax.fori_loop(0, rp // LANES, fill, 0)
    pltpu.sync_copy(zvec, deg_acc.at[pl.ds(s * rp, rp)])
    plsc.subcore_barrier()

    def chunk(k, _):
        pltpu.sync_copy(ones_v, deg_acc.at[didx.at[k]], add=True)
        return 0

    lax.fori_loop(0, ep // KD, chunk, 0)
    plsc.subcore_barrier()
    pltpu.sync_copy(deg_acc.at[pl.ds(s * rp, rp)], out.at[c, pl.ds(s * rp, rp)])


def _agg_body(n_acc, ept, d, g, eidx, out, ebuf, rows, acc):
    c = lax.axis_index("c")
    s = lax.axis_index("s")
    wid = c * NS + s
    nch = ept // KD
    arp = n_acc // NS

    def fill(i, _):
        rows[i // (d // LANES), pl.ds((i % (d // LANES)) * LANES, LANES)] = (
            jnp.zeros((LANES,), jnp.float32)
        )
        return 0

    lax.fori_loop(0, KD * (d // LANES), fill, 0)
    for z in range(arp // KD):
        pltpu.sync_copy(
            rows, acc.at[pl.ds(pl.multiple_of(s * arp + z * KD, 8), KD)]
        )
    plsc.subcore_barrier()

    def chunk(k, _):
        off = pl.multiple_of((wid * nch + k) * 2 * KD, 8)
        pltpu.sync_copy(eidx.at[pl.ds(off, 2 * KD)], ebuf)
        pltpu.sync_copy(g.at[ebuf.at[pl.ds(0, KD)]], rows)
        pltpu.sync_copy(rows, acc.at[ebuf.at[pl.ds(KD, KD)]], add=True)
        return 0

    lax.fori_loop(0, nch, chunk, 0)
    plsc.subcore_barrier()
    aoff = pl.multiple_of(s * arp, 8)
    pltpu.sync_copy(acc.at[pl.ds(aoff, arp)], out.at[c, pl.ds(aoff, arp)])


def _tca_body(x_ref, w_ref, degp_ref, g_ref):
    deg = degp_ref[0, :] + degp_ref[1, :] + 1.0
    dinv = lax.rsqrt(deg)[:, None]
    h = jnp.dot(x_ref[...], w_ref[...], preferred_element_type=jnp.float32)
    g_ref[...] = h * dinv


def _tcb_body(p_ref, g_ref, degp_ref, w_ref, b_ref, out_ref):
    deg = degp_ref[0, :] + degp_ref[1, :] + 1.0
    dinv = lax.rsqrt(deg)[:, None]
    t = dinv * (p_ref[0] + p_ref[1] + g_ref[...]) + b_ref[...]
    o1 = jnp.where(t >= 0, t, 0.01 * t)
    h2 = jnp.dot(o1, w_ref[...], preferred_element_type=jnp.float32)
    out_ref[...] = h2 * dinv


def _tcc_body(p_ref, g_ref, degp_ref, b_ref, out_ref):
    deg = degp_ref[0, :] + degp_ref[1, :] + 1.0
    dinv = lax.rsqrt(deg)[:, None]
    t = dinv * (p_ref[0] + p_ref[1] + g_ref[...]) + b_ref[...]
    out_ref[...] = jnp.where(t >= 0, t, 0.01 * t)


def kernel(x, edge_index, batch, W1, b1, W2, b2):
    n, d = x.shape
    e = edge_index.shape[1]

    n_acc = -(-n // (NS * KD)) * NS * KD

    ept = -(-e // (NW * KD)) * KD
    e_pad = ept * NW
    nch = ept // KD
    src = edge_index[0].astype(jnp.int32)
    dst = edge_index[1].astype(jnp.int32)
    pad = e_pad - e
    srcp = jnp.concatenate([src, jnp.zeros((pad,), jnp.int32)])
    dstp = jnp.concatenate([dst, jnp.full((pad,), n, jnp.int32)])
    eidx = jnp.stack(
        [srcp.reshape(NW, nch, KD), dstp.reshape(NW, nch, KD)], axis=2
    ).reshape(-1)

    epd = -(-e // (NW * KD)) * KD
    e_pad_d = epd * NW
    pad_d = e_pad_d - e
    dst3 = jnp.concatenate([dst, jnp.full((pad_d,), n, jnp.int32)]).reshape(
        NW, epd // KD, KD
    )

    xp = jnp.concatenate([x, jnp.zeros((n_acc - n, d), x.dtype)])

    deg_kernel = pl.kernel(
        functools.partial(_deg_body, n_acc, epd),
        out_type=jax.ShapeDtypeStruct((NC, n_acc), jnp.float32),
        mesh=_mesh(),
        scratch_types={
            "didx": pltpu.VMEM((epd // KD, KD), jnp.int32),
            "ones_v": pltpu.VMEM((KD,), jnp.float32),
            "zvec": pltpu.VMEM((n_acc // NS,), jnp.float32),
            "deg_acc": pltpu.MemorySpace.VMEM_SHARED((n_acc,), jnp.float32),
        },
        name="gcn_sc_degree",
    )

    agg_kernel = pl.kernel(
        functools.partial(_agg_body, n_acc, ept, d),
        out_type=jax.ShapeDtypeStruct((NC, n_acc, d), jnp.float32),
        mesh=_mesh(),
        scratch_types={
            "ebuf": pltpu.VMEM((2 * KD,), jnp.int32),
            "rows": pltpu.VMEM((KD, d), jnp.float32),
            "acc": pltpu.MemorySpace.VMEM_SHARED((n_acc, d), jnp.float32),
        },
        name="gcn_sc_scatter",
    )

    br = 2048
    grid = (n_acc // br,)
    tca = pl.pallas_call(
        _tca_body,
        grid=grid,
        in_specs=[
            pl.BlockSpec((br, d), lambda i: (i, 0)),
            pl.BlockSpec((d, d), lambda i: (0, 0)),
            pl.BlockSpec((NC, br), lambda i: (0, i)),
        ],
        out_specs=pl.BlockSpec((br, d), lambda i: (i, 0)),
        out_shape=jax.ShapeDtypeStruct((n_acc, d), jnp.float32),
        name="gcn_tc_g1",
    )
    tcb = pl.pallas_call(
        _tcb_body,
        grid=grid,
        in_specs=[
            pl.BlockSpec((NC, br, d), lambda i: (0, i, 0)),
            pl.BlockSpec((br, d), lambda i: (i, 0)),
            pl.BlockSpec((NC, br), lambda i: (0, i)),
            pl.BlockSpec((d, d), lambda i: (0, 0)),
            pl.BlockSpec((1, d), lambda i: (0, 0)),
        ],
        out_specs=pl.BlockSpec((br, d), lambda i: (i, 0)),
        out_shape=jax.ShapeDtypeStruct((n_acc, d), jnp.float32),
        name="gcn_tc_layer1",
    )
    tcc = pl.pallas_call(
        _tcc_body,
        grid=grid,
        in_specs=[
            pl.BlockSpec((NC, br, d), lambda i: (0, i, 0)),
            pl.BlockSpec((br, d), lambda i: (i, 0)),
            pl.BlockSpec((NC, br), lambda i: (0, i)),
            pl.BlockSpec((1, d), lambda i: (0, 0)),
        ],
        out_specs=pl.BlockSpec((br, d), lambda i: (i, 0)),
        out_shape=jax.ShapeDtypeStruct((n_acc, d), jnp.float32),
        name="gcn_tc_layer2",
    )

    degp = deg_kernel(dst3)
    g1 = tca(xp, W1, degp)
    p1 = agg_kernel(g1, eidx)
    g2 = tcb(p1, g1, degp, W2, b1.reshape(1, d))
    p2 = agg_kernel(g2, eidx)
    out = tcc(p2, g2, degp, b2.reshape(1, d))
    return out[:n]
